# baseline (speedup 1.0000x reference)
import functools

import jax
import jax.numpy as jnp
from jax.experimental import pallas as pl
from jax.experimental.pallas import tpu as pltpu

_CHUNKS = 8


def _dma_copy_scatter_kernel(idx_ref, x_ref, copy_ref, out_ref, sems, row_sem,
                             *, chunks, chunk_rows):
    for c in range(chunks):
        sl = pl.ds(c * chunk_rows, chunk_rows)
        pltpu.make_async_copy(x_ref.at[sl], out_ref.at[sl], sems.at[c]).start()
    for c in range(chunks):
        sl = pl.ds(c * chunk_rows, chunk_rows)
        pltpu.make_async_copy(x_ref.at[sl], out_ref.at[sl], sems.at[c]).wait()
    idx = idx_ref[0]
    row = pltpu.make_async_copy(copy_ref, out_ref.at[pl.ds(idx, 1), :], row_sem)
    row.start()
    row.wait()


def kernel(x, copy_tensor, indices):
    rows, cols = x.shape
    chunks = _CHUNKS if rows % _CHUNKS == 0 else 1
    chunk_rows = rows // chunks
    return pl.pallas_call(
        functools.partial(_dma_copy_scatter_kernel, chunks=chunks,
                          chunk_rows=chunk_rows),
        in_specs=[
            pl.BlockSpec(memory_space=pltpu.SMEM),
            pl.BlockSpec(memory_space=pl.ANY),
            pl.BlockSpec(memory_space=pl.ANY),
        ],
        out_specs=pl.BlockSpec(memory_space=pl.ANY),
        out_shape=jax.ShapeDtypeStruct((rows, cols), x.dtype),
        scratch_shapes=[pltpu.SemaphoreType.DMA((_CHUNKS,)),
                        pltpu.SemaphoreType.DMA],
    )(indices, x, copy_tensor)


# speedup vs baseline: 11.8081x; 11.8081x over previous
import functools

import jax
import jax.numpy as jnp
from jax.experimental import pallas as pl
from jax.experimental.pallas import tpu as pltpu

_BLOCK_ROWS = 10000


def _copy_scatter_wide_kernel(idx_ref, x_ref, copy_ref, out_ref, *, block_rows,
                              cols):
    i = pl.program_id(0)
    idx = idx_ref[0]
    wide = 2 * cols
    wrow = idx // 2
    base = i * block_rows
    hit = (wrow >= base) & (wrow < base + block_rows)

    @pl.when(jnp.logical_not(hit))
    def _miss():
        out_ref[...] = x_ref[...]

    @pl.when(hit)
    def _hit():
        lane0 = (idx % 2) * cols
        row_ids = jax.lax.broadcasted_iota(jnp.int32, (block_rows, wide), 0)
        lane_ids = jax.lax.broadcasted_iota(jnp.int32, (block_rows, wide), 1)
        mask = (row_ids == (wrow - base)) & (lane_ids >= lane0) & \
               (lane_ids < lane0 + cols)
        ct2 = jnp.concatenate([copy_ref[...], copy_ref[...]], axis=-1)
        out_ref[...] = jnp.where(mask, ct2, x_ref[...])


def kernel(x, copy_tensor, indices):
    rows, cols = x.shape
    if cols == 64 and rows % 2 == 0 and (rows // 2) % _BLOCK_ROWS == 0:
        wrows, wide = rows // 2, 2 * cols
        xw = x.reshape(wrows, wide)
        outw = pl.pallas_call(
            functools.partial(_copy_scatter_wide_kernel,
                              block_rows=_BLOCK_ROWS, cols=cols),
            grid_spec=pltpu.PrefetchScalarGridSpec(
                num_scalar_prefetch=1,
                grid=(wrows // _BLOCK_ROWS,),
                in_specs=[
                    pl.BlockSpec((_BLOCK_ROWS, wide), lambda i, idx: (i, 0)),
                    pl.BlockSpec((1, cols), lambda i, idx: (0, 0)),
                ],
                out_specs=pl.BlockSpec((_BLOCK_ROWS, wide),
                                       lambda i, idx: (i, 0)),
            ),
            out_shape=jax.ShapeDtypeStruct((wrows, wide), x.dtype),
        )(indices, xw, copy_tensor)
        return outw.reshape(rows, cols)
    block_rows = next(b for b in (8000, 5000, 2000, 1000, 8, 1)
                      if rows % b == 0)
    return pl.pallas_call(
        functools.partial(_copy_scatter_narrow_kernel, block_rows=block_rows),
        grid_spec=pltpu.PrefetchScalarGridSpec(
            num_scalar_prefetch=1,
            grid=(rows // block_rows,),
            in_specs=[
                pl.BlockSpec((block_rows, cols), lambda i, idx: (i, 0)),
                pl.BlockSpec((1, cols), lambda i, idx: (0, 0)),
            ],
            out_specs=pl.BlockSpec((block_rows, cols), lambda i, idx: (i, 0)),
        ),
        out_shape=jax.ShapeDtypeStruct((rows, cols), x.dtype),
    )(indices, x, copy_tensor)


def _copy_scatter_narrow_kernel(idx_ref, x_ref, copy_ref, out_ref, *,
                                block_rows):
    i = pl.program_id(0)
    out_ref[...] = x_ref[...]
    idx = idx_ref[0]
    base = i * block_rows
    @pl.when((idx >= base) & (idx < base + block_rows))
    def _():
        out_ref[pl.ds(idx - base, 1), :] = copy_ref[...]


# speedup vs baseline: 16.1297x; 1.3660x over previous
import functools

import jax
import jax.numpy as jnp
from jax.experimental import pallas as pl
from jax.experimental.pallas import tpu as pltpu

_WIDE_FACTOR = 8
_CHUNK_ROWS = 2500


def _copy_scatter_kernel(idx_ref, copy_ref, x_hbm, out_hbm, row_sem, *,
                         wrows, wide, chunk_rows):
    group = wide // 64
    xw = x_hbm.reshape(wrows, group, 64)
    ow = out_hbm.reshape(wrows, group, 64)

    def body(xb, ob):
        ob[...] = xb[...]

    pltpu.emit_pipeline(
        body,
        grid=(wrows // chunk_rows,),
        in_specs=[pl.BlockSpec((chunk_rows, group, 64), lambda i: (i, 0, 0))],
        out_specs=[pl.BlockSpec((chunk_rows, group, 64), lambda i: (i, 0, 0))],
    )(xw, ow)

    idx = idx_ref[0]
    row = pltpu.make_async_copy(copy_ref, out_hbm.at[pl.ds(idx, 1), :], row_sem)
    row.start()
    row.wait()


def kernel(x, copy_tensor, indices):
    rows, cols = x.shape
    wide = _WIDE_FACTOR * cols
    assert rows % _WIDE_FACTOR == 0
    wrows = rows // _WIDE_FACTOR
    assert wrows % _CHUNK_ROWS == 0
    return pl.pallas_call(
        functools.partial(_copy_scatter_kernel, wrows=wrows, wide=wide,
                          chunk_rows=_CHUNK_ROWS),
        in_specs=[
            pl.BlockSpec(memory_space=pltpu.SMEM),
            pl.BlockSpec((1, cols), lambda: (0, 0)),
            pl.BlockSpec(memory_space=pl.ANY),
        ],
        out_specs=pl.BlockSpec(memory_space=pl.ANY),
        out_shape=jax.ShapeDtypeStruct((rows, cols), x.dtype),
        scratch_shapes=[pltpu.SemaphoreType.DMA],
    )(indices, copy_tensor, x)


# speedup vs baseline: 23.6363x; 1.4654x over previous
import jax
import jax.numpy as jnp
from jax.experimental import pallas as pl
from jax.experimental.pallas import tpu as pltpu


def _scatter_row_kernel(idx_ref, copy_ref, x_hbm, out_hbm, sem):
    del x_hbm
    idx = idx_ref[0]
    row = pltpu.make_async_copy(copy_ref, out_hbm.at[pl.ds(idx, 1), :], sem)
    row.start()
    row.wait()


def kernel(x, copy_tensor, indices):
    rows, cols = x.shape
    return pl.pallas_call(
        _scatter_row_kernel,
        in_specs=[
            pl.BlockSpec(memory_space=pltpu.SMEM),
            pl.BlockSpec((1, cols), lambda: (0, 0)),
            pl.BlockSpec(memory_space=pl.ANY),
        ],
        out_specs=pl.BlockSpec(memory_space=pl.ANY),
        out_shape=jax.ShapeDtypeStruct((rows, cols), x.dtype),
        input_output_aliases={2: 0},
        scratch_shapes=[pltpu.SemaphoreType.DMA],
    )(indices, copy_tensor, x)


# speedup vs baseline: 23.7136x; 1.0033x over previous
import jax
import jax.numpy as jnp
from jax.experimental import pallas as pl
from jax.experimental.pallas import tpu as pltpu


def _scatter_row_kernel(idx_ref, copy_ref, x_hbm, out_hbm, sem):
    del x_hbm
    idx = idx_ref[0]
    row = pltpu.make_async_copy(copy_ref, out_hbm.at[pl.ds(idx, 1), :], sem)
    row.start()
    row.wait()


def kernel(x, copy_tensor, indices):
    rows, cols = x.shape
    x = jnp.copy(x)
    return pl.pallas_call(
        _scatter_row_kernel,
        in_specs=[
            pl.BlockSpec(memory_space=pltpu.SMEM),
            pl.BlockSpec((1, cols), lambda: (0, 0)),
            pl.BlockSpec(memory_space=pl.ANY),
        ],
        out_specs=pl.BlockSpec(memory_space=pl.ANY),
        out_shape=jax.ShapeDtypeStruct((rows, cols), x.dtype),
        input_output_aliases={2: 0},
        scratch_shapes=[pltpu.SemaphoreType.DMA],
    )(indices, copy_tensor, x)
